# baseline (speedup 1.0000x reference)
import functools

import jax
import jax.numpy as jnp
from jax import lax
from jax.experimental import pallas as pl
from jax.experimental.pallas import tpu as pltpu
from jax.experimental.pallas import tpu_sc as plsc

N = 320000
D_IN = 128
D_OUT = 64
NSEG = 10000
EPS = 1e-3

NH = N // 2
DW = 2 * D_OUT


RAH = 3200


def _mm_body(x1_ref, x2_ref, wt_ref, gamma_ref, beta_ref, o_ref, ab_ref,
             acc_ref):
    i = pl.program_id(0)

    @pl.when(i == 0)
    def _():
        acc_ref[...] = jnp.zeros_like(acc_ref)

    y1 = jnp.dot(x1_ref[...], wt_ref[...], preferred_element_type=jnp.float32)
    y2 = jnp.dot(x2_ref[...], wt_ref[...], preferred_element_type=jnp.float32)
    o_ref[...] = jnp.concatenate([y1, y2], axis=1)
    acc_ref[0:1, :] += (jnp.sum(y1, axis=0, keepdims=True)
                        + jnp.sum(y2, axis=0, keepdims=True))
    acc_ref[1:2, :] += (jnp.sum(y1 * y1, axis=0, keepdims=True)
                        + jnp.sum(y2 * y2, axis=0, keepdims=True))

    @pl.when(i == pl.num_programs(0) - 1)
    def _():
        mean = acc_ref[0, :] * (1.0 / N)
        var = acc_ref[1, :] * (1.0 / N) - mean * mean
        a = gamma_ref[0, :] * lax.rsqrt(var + EPS)
        b = beta_ref[0, :] - mean * a
        ab_ref[...] = jnp.stack(
            [jnp.concatenate([a, a]), jnp.concatenate([b, b])], axis=0)


def _matmul_coeffs(inputs, wt, gamma, beta):
    nb = NH // RAH
    return pl.pallas_call(
        _mm_body,
        grid=(nb,),
        in_specs=[
            pl.BlockSpec((RAH, D_IN), lambda i: (i, 0)),
            pl.BlockSpec((RAH, D_IN), lambda i: (i + nb, 0)),
            pl.BlockSpec((D_IN, D_OUT), lambda i: (0, 0)),
            pl.BlockSpec((1, D_OUT), lambda i: (0, 0)),
            pl.BlockSpec((1, D_OUT), lambda i: (0, 0)),
        ],
        out_specs=[
            pl.BlockSpec((RAH, DW), lambda i: (i, 0)),
            pl.BlockSpec((2, DW), lambda i: (0, 0)),
        ],
        out_shape=[
            jax.ShapeDtypeStruct((NH, DW), jnp.float32),
            jax.ShapeDtypeStruct((2, DW), jnp.float32),
        ],
        scratch_shapes=[pltpu.VMEM((2, D_OUT), jnp.float32)],
    )(inputs, inputs, wt, gamma, beta)


BLK = 128
BLKH = BLK // 2
NBLK = N // BLK
NTILES = 32
NSEG_PAD = 10240
STRIPE = NSEG_PAD // 16
CW = 16

_sc_mesh = plsc.VectorSubcoreMesh(core_axis_name="c", subcore_axis_name="s")
_sc_params = pltpu.CompilerParams(use_tc_tiling_on_sc=False)


def _zero_rows(buf, nrows, ncols):
    z = jnp.zeros((16,), jnp.float32)

    @pl.loop(0, nrows)
    def _(r):
        for c in range(ncols // 16):
            buf[r, pl.ds(16 * c, 16)] = z


def _load_coeffs(abuf):
    a_regs = [abuf[0, pl.ds(16 * c, 16)] for c in range(8)]
    b_regs = [abuf[1, pl.ds(16 * c, 16)] for c in range(8)]
    return a_regs, b_regs


def _normalize_block(dbuf, nbuf, a_regs, b_regs):
    @plsc.parallel_loop(0, BLKH, unroll=8)
    def _(r):
        for c in range(8):
            v = dbuf[r, pl.ds(16 * c, 16)]
            v = jnp.maximum(v * a_regs[c] + b_regs[c], jnp.float32(0.0))
            nbuf[r + BLKH * (c // 4), pl.ds((c % 4) * 16, 16)] = v


def _issue_loads(x_hbm, inv_hbm, b, dbuf, ibuf, semd, semi):
    pltpu.async_copy(x_hbm.at[pl.ds(b * BLKH, BLKH)], dbuf, semd)
    pltpu.async_copy(inv_hbm.at[pl.ds(b * BLKH, BLKH)],
                     ibuf.at[pl.ds(0, BLKH)], semi)
    pltpu.async_copy(inv_hbm.at[pl.ds(NH + b * BLKH, BLKH)],
                     ibuf.at[pl.ds(BLKH, BLKH)], semi)


def _wait_loads(x_hbm, inv_hbm, b, dbuf, ibuf, semd, semi):
    pltpu.make_async_copy(x_hbm.at[pl.ds(b * BLKH, BLKH)], dbuf, semd).wait()
    pltpu.make_async_copy(inv_hbm.at[pl.ds(b * BLKH, BLKH)],
                          ibuf.at[pl.ds(0, BLKH)], semi).wait()
    pltpu.make_async_copy(inv_hbm.at[pl.ds(NH + b * BLKH, BLKH)],
                          ibuf.at[pl.ds(BLKH, BLKH)], semi).wait()


@functools.partial(
    pl.kernel,
    mesh=_sc_mesh,
    out_type=(
        jax.ShapeDtypeStruct((2, NSEG_PAD, D_OUT), jnp.float32),
        jax.ShapeDtypeStruct((2, NSEG_PAD, CW), jnp.float32),
    ),
    scratch_types=[
        pltpu.VMEM((2, BLKH, DW), jnp.float32),
        pltpu.VMEM((2, BLK, D_OUT), jnp.float32),
        pltpu.VMEM((2, BLK), jnp.int32),
        pltpu.VMEM((BLK, CW), jnp.float32),
        pltpu.VMEM((2, DW), jnp.float32),
        pltpu.VMEM_SHARED((NSEG_PAD, D_OUT), jnp.float32),
        pltpu.VMEM_SHARED((NSEG_PAD, CW), jnp.float32),
        pltpu.SemaphoreType.DMA((2,)),
        pltpu.SemaphoreType.DMA((2,)),
    ],
    compiler_params=_sc_params,
)
def _segsum(x_hbm, inv_hbm, ab_hbm, osum_hbm, ocnt_hbm,
            dbuf2, nbuf2, ibuf2, ones, abuf, tsum, tcnt,
            semd, semi):
    cid = lax.axis_index("c")
    sid = lax.axis_index("s")
    wid = sid * 2 + cid

    pltpu.sync_copy(ab_hbm, abuf)
    a_regs, b_regs = _load_coeffs(abuf)

    zrows = nbuf2.at[0]
    _zero_rows(zrows, BLK, D_OUT)
    _zero_rows(ones, BLK, CW)
    for j in range(STRIPE // BLK):
        pltpu.sync_copy(zrows,
                        tsum.at[pl.ds(sid * STRIPE + j * BLK, BLK)])
        pltpu.sync_copy(ones,
                        tcnt.at[pl.ds(sid * STRIPE + j * BLK, BLK)])
    onerow = jnp.full((16,), 1.0, jnp.float32)

    @pl.loop(0, BLK)
    def _(r):
        ones[r, pl.ds(0, 16)] = onerow

    plsc.subcore_barrier()

    bufs = [(dbuf2.at[p], nbuf2.at[p], ibuf2.at[p], semd.at[p], semi.at[p])
            for p in range(2)]

    for p in range(2):
        b = wid + p * NTILES
        dbuf, nbuf, ibuf, sd, si = bufs[p]

        @pl.when(b < NBLK)
        def _():
            _issue_loads(x_hbm, inv_hbm, b, dbuf, ibuf, sd, si)

    def _phase(k, p):
        b = wid + (k + p) * NTILES
        dbuf, nbuf, ibuf, sd, si = bufs[p]

        @pl.when(b < NBLK)
        def _():
            _wait_loads(x_hbm, inv_hbm, b, dbuf, ibuf, sd, si)
            _normalize_block(dbuf, nbuf, a_regs, b_regs)
            pltpu.sync_copy(nbuf, tsum.at[ibuf], add=True)
            pltpu.sync_copy(ones, tcnt.at[ibuf], add=True)
            bn = b + 2 * NTILES

            @pl.when(bn < NBLK)
            def _():
                _issue_loads(x_hbm, inv_hbm, bn, dbuf, ibuf, sd, si)

    @pl.loop(0, 80, step=2)
    def _(k):
        _phase(k, 0)
        _phase(k, 1)

    plsc.subcore_barrier()
    pltpu.sync_copy(tsum.at[pl.ds(sid * STRIPE, STRIPE)],
                    osum_hbm.at[cid, pl.ds(sid * STRIPE, STRIPE)])
    pltpu.sync_copy(tcnt.at[pl.ds(sid * STRIPE, STRIPE)],
                    ocnt_hbm.at[cid, pl.ds(sid * STRIPE, STRIPE)])


MSTR = NSEG_PAD // NTILES


@functools.partial(
    pl.kernel,
    mesh=_sc_mesh,
    out_type=jax.ShapeDtypeStruct((NSEG_PAD, D_OUT), jnp.float32),
    scratch_types=[
        pltpu.VMEM((2, MSTR, D_OUT), jnp.float32),
        pltpu.VMEM((2, MSTR, CW), jnp.float32),
        pltpu.VMEM((MSTR, D_OUT), jnp.float32),
    ],
    compiler_params=_sc_params,
)
def _means(psum_hbm, pcnt_hbm, o_hbm, sbuf, cbuf, obuf):
    cid = lax.axis_index("c")
    sid = lax.axis_index("s")
    wid = sid * 2 + cid
    base = wid * MSTR
    for h in range(2):
        pltpu.sync_copy(psum_hbm.at[h, pl.ds(base, MSTR)], sbuf.at[h])
        pltpu.sync_copy(pcnt_hbm.at[h, pl.ds(base, MSTR)], cbuf.at[h])

    one = jnp.full((16,), 1.0, jnp.float32)

    @plsc.parallel_loop(0, MSTR, unroll=4)
    def _(r):
        cnt = cbuf[0, r, pl.ds(0, 16)] + cbuf[1, r, pl.ds(0, 16)]
        recip = one / jnp.maximum(cnt, one)
        for c in range(4):
            s = (sbuf[0, r, pl.ds(16 * c, 16)]
                 + sbuf[1, r, pl.ds(16 * c, 16)])
            obuf[r, pl.ds(16 * c, 16)] = s * recip

    pltpu.sync_copy(obuf, o_hbm.at[pl.ds(base, MSTR)])


def _normalize_to_out(dbuf, obuf, a_regs, b_regs):
    @plsc.parallel_loop(0, BLKH, unroll=8)
    def _(r):
        for c in range(8):
            v = dbuf[r, pl.ds(16 * c, 16)]
            v = jnp.maximum(v * a_regs[c] + b_regs[c], jnp.float32(0.0))
            obuf[r + BLKH * (c // 4), pl.ds((c % 4) * 16, 16)] = v


def _merge_means(gbuf, obuf):
    @plsc.parallel_loop(0, BLK, unroll=8)
    def _(r):
        for c in range(4):
            obuf[r, pl.ds(D_OUT + 16 * c, 16)] = gbuf[r, pl.ds(16 * c, 16)]


@functools.partial(
    pl.kernel,
    mesh=_sc_mesh,
    out_type=jax.ShapeDtypeStruct((N, DW), jnp.float32),
    scratch_types=[
        pltpu.VMEM((2, BLKH, DW), jnp.float32),
        pltpu.VMEM((2, BLK, DW), jnp.float32),
        pltpu.VMEM((2, BLK, D_OUT), jnp.float32),
        pltpu.VMEM((2, BLK), jnp.int32),
        pltpu.VMEM((2, DW), jnp.float32),
        pltpu.VMEM_SHARED((NSEG_PAD, D_OUT), jnp.float32),
        pltpu.SemaphoreType.DMA((2,)),
        pltpu.SemaphoreType.DMA((2,)),
        pltpu.SemaphoreType.DMA((2,)),
        pltpu.SemaphoreType.DMA((2,)),
    ],
    compiler_params=_sc_params,
)
def _gather_out(x_hbm, inv_hbm, ab_hbm, means_hbm, out_hbm,
                dbuf2, obuf2, gbuf2, ibuf2, abuf, smeans,
                semd, semi, semg, semw):
    cid = lax.axis_index("c")
    sid = lax.axis_index("s")
    wid = sid * 2 + cid

    pltpu.sync_copy(means_hbm.at[pl.ds(sid * STRIPE, STRIPE)],
                    smeans.at[pl.ds(sid * STRIPE, STRIPE)])
    pltpu.sync_copy(ab_hbm, abuf)
    a_regs, b_regs = _load_coeffs(abuf)
    plsc.subcore_barrier()

    bufs = [(dbuf2.at[p], obuf2.at[p], gbuf2.at[p], ibuf2.at[p],
             semd.at[p], semi.at[p], semg.at[p], semw.at[p])
            for p in range(2)]

    for p in range(2):
        b = wid + p * NTILES
        dbuf, obuf, gbuf, ibuf, sd, si, sg, sw = bufs[p]

        @pl.when(b < NBLK)
        def _():
            _issue_loads(x_hbm, inv_hbm, b, dbuf, ibuf, sd, si)

    def _wait_writes(p, b):
        dbuf, obuf, gbuf, ibuf, sd, si, sg, sw = bufs[p]
        pltpu.make_async_copy(obuf.at[pl.ds(0, BLKH)],
                              out_hbm.at[pl.ds(b * BLKH, BLKH)], sw).wait()
        pltpu.make_async_copy(obuf.at[pl.ds(BLKH, BLKH)],
                              out_hbm.at[pl.ds(NH + b * BLKH, BLKH)], sw).wait()

    def _phase(k, p):
        b = wid + (k + p) * NTILES
        dbuf, obuf, gbuf, ibuf, sd, si, sg, sw = bufs[p]

        @pl.when(b < NBLK)
        def _():
            _wait_loads(x_hbm, inv_hbm, b, dbuf, ibuf, sd, si)
            pltpu.async_copy(smeans.at[ibuf], gbuf, sg)

            @pl.when(k + p >= 2)
            def _():
                _wait_writes(p, b)

            _normalize_to_out(dbuf, obuf, a_regs, b_regs)
            pltpu.make_async_copy(smeans.at[ibuf], gbuf, sg).wait()
            _merge_means(gbuf, obuf)
            pltpu.async_copy(obuf.at[pl.ds(0, BLKH)],
                             out_hbm.at[pl.ds(b * BLKH, BLKH)], sw)
            pltpu.async_copy(obuf.at[pl.ds(BLKH, BLKH)],
                             out_hbm.at[pl.ds(NH + b * BLKH, BLKH)], sw)
            bn = b + 2 * NTILES

            @pl.when(bn < NBLK)
            def _():
                _issue_loads(x_hbm, inv_hbm, bn, dbuf, ibuf, sd, si)

    @pl.loop(0, 80, step=2)
    def _(k):
        _phase(k, 0)
        _phase(k, 1)

    for p in range(2):
        b = wid + p * NTILES

        @pl.when(b < NBLK)
        def _():
            _wait_writes(p, b)


def kernel(inputs, unq_inv, W, gamma, beta):
    wt = W.T
    g2 = gamma.reshape(1, D_OUT)
    b2 = beta.reshape(1, D_OUT)
    x, ab = _matmul_coeffs(inputs, wt, g2, b2)
    psum, pcnt = _segsum(x, unq_inv, ab)
    means = _means(psum, pcnt)
    return _gather_out(x, unq_inv, ab, means)


# speedup vs baseline: 1.0291x; 1.0291x over previous
import functools

import jax
import jax.numpy as jnp
from jax import lax
from jax.experimental import pallas as pl
from jax.experimental.pallas import tpu as pltpu
from jax.experimental.pallas import tpu_sc as plsc

N = 320000
D_IN = 128
D_OUT = 64
NSEG = 10000
EPS = 1e-3

NH = N // 2
DW = 2 * D_OUT


RAH = 3200


def _mm_body(x1_ref, x2_ref, wt_ref, gamma_ref, beta_ref, o_ref, ab_ref,
             acc_ref):
    i = pl.program_id(0)

    @pl.when(i == 0)
    def _():
        acc_ref[...] = jnp.zeros_like(acc_ref)

    y1 = jnp.dot(x1_ref[...], wt_ref[...], preferred_element_type=jnp.float32)
    y2 = jnp.dot(x2_ref[...], wt_ref[...], preferred_element_type=jnp.float32)
    o_ref[...] = jnp.concatenate([y1, y2], axis=1)
    acc_ref[0:1, :] += (jnp.sum(y1, axis=0, keepdims=True)
                        + jnp.sum(y2, axis=0, keepdims=True))
    acc_ref[1:2, :] += (jnp.sum(y1 * y1, axis=0, keepdims=True)
                        + jnp.sum(y2 * y2, axis=0, keepdims=True))

    @pl.when(i == pl.num_programs(0) - 1)
    def _():
        mean = acc_ref[0, :] * (1.0 / N)
        var = acc_ref[1, :] * (1.0 / N) - mean * mean
        a = gamma_ref[0, :] * lax.rsqrt(var + EPS)
        b = beta_ref[0, :] - mean * a
        ab_ref[...] = jnp.stack(
            [jnp.concatenate([a, a]), jnp.concatenate([b, b])], axis=0)


def _matmul_coeffs(inputs, wt, gamma, beta):
    nb = NH // RAH
    return pl.pallas_call(
        _mm_body,
        grid=(nb,),
        in_specs=[
            pl.BlockSpec((RAH, D_IN), lambda i: (i, 0)),
            pl.BlockSpec((RAH, D_IN), lambda i: (i + nb, 0)),
            pl.BlockSpec((D_IN, D_OUT), lambda i: (0, 0)),
            pl.BlockSpec((1, D_OUT), lambda i: (0, 0)),
            pl.BlockSpec((1, D_OUT), lambda i: (0, 0)),
        ],
        out_specs=[
            pl.BlockSpec((RAH, DW), lambda i: (i, 0)),
            pl.BlockSpec((2, DW), lambda i: (0, 0)),
        ],
        out_shape=[
            jax.ShapeDtypeStruct((NH, DW), jnp.float32),
            jax.ShapeDtypeStruct((2, DW), jnp.float32),
        ],
        scratch_shapes=[pltpu.VMEM((2, D_OUT), jnp.float32)],
    )(inputs, inputs, wt, gamma, beta)


BLK = 128
BLKH = BLK // 2
NBLK = N // BLK
NTILES = 32
NSEG_PAD = 10240
STRIPE = NSEG_PAD // 16
TW = D_OUT + 16

_sc_mesh = plsc.VectorSubcoreMesh(core_axis_name="c", subcore_axis_name="s")
_sc_params = pltpu.CompilerParams(use_tc_tiling_on_sc=False)


def _zero_rows(buf, nrows, ncols):
    z = jnp.zeros((16,), jnp.float32)

    @pl.loop(0, nrows)
    def _(r):
        for c in range(ncols // 16):
            buf[r, pl.ds(16 * c, 16)] = z


def _load_coeffs(abuf):
    a_regs = [abuf[0, pl.ds(16 * c, 16)] for c in range(8)]
    b_regs = [abuf[1, pl.ds(16 * c, 16)] for c in range(8)]
    return a_regs, b_regs


def _normalize_block(dbuf, nbuf, a_regs, b_regs):
    @plsc.parallel_loop(0, BLKH, unroll=8)
    def _(r):
        for c in range(8):
            v = dbuf[r, pl.ds(16 * c, 16)]
            v = jnp.maximum(v * a_regs[c] + b_regs[c], jnp.float32(0.0))
            nbuf[r + BLKH * (c // 4), pl.ds((c % 4) * 16, 16)] = v


def _normalize_block_cnt(dbuf, nbuf, a_regs, b_regs):
    one = jnp.full((16,), 1.0, jnp.float32)

    @plsc.parallel_loop(0, BLKH, unroll=8)
    def _(r):
        for c in range(8):
            v = dbuf[r, pl.ds(16 * c, 16)]
            v = jnp.maximum(v * a_regs[c] + b_regs[c], jnp.float32(0.0))
            nbuf[r + BLKH * (c // 4), pl.ds((c % 4) * 16, 16)] = v
        nbuf[r, pl.ds(D_OUT, 16)] = one
        nbuf[r + BLKH, pl.ds(D_OUT, 16)] = one


def _issue_loads(x_hbm, inv_hbm, b, dbuf, ibuf, semd, semi):
    pltpu.async_copy(x_hbm.at[pl.ds(b * BLKH, BLKH)], dbuf, semd)
    pltpu.async_copy(inv_hbm.at[pl.ds(b * BLKH, BLKH)],
                     ibuf.at[pl.ds(0, BLKH)], semi)
    pltpu.async_copy(inv_hbm.at[pl.ds(NH + b * BLKH, BLKH)],
                     ibuf.at[pl.ds(BLKH, BLKH)], semi)


def _wait_loads(x_hbm, inv_hbm, b, dbuf, ibuf, semd, semi):
    pltpu.make_async_copy(x_hbm.at[pl.ds(b * BLKH, BLKH)], dbuf, semd).wait()
    pltpu.make_async_copy(inv_hbm.at[pl.ds(b * BLKH, BLKH)],
                          ibuf.at[pl.ds(0, BLKH)], semi).wait()
    pltpu.make_async_copy(inv_hbm.at[pl.ds(NH + b * BLKH, BLKH)],
                          ibuf.at[pl.ds(BLKH, BLKH)], semi).wait()


@functools.partial(
    pl.kernel,
    mesh=_sc_mesh,
    out_type=jax.ShapeDtypeStruct((2, NSEG_PAD, TW), jnp.float32),
    scratch_types=[
        pltpu.VMEM((2, BLKH, DW), jnp.float32),
        pltpu.VMEM((2, BLK, TW), jnp.float32),
        pltpu.VMEM((2, BLK), jnp.int32),
        pltpu.VMEM((2, DW), jnp.float32),
        pltpu.VMEM_SHARED((NSEG_PAD, TW), jnp.float32),
        pltpu.SemaphoreType.DMA((2,)),
        pltpu.SemaphoreType.DMA((2,)),
    ],
    compiler_params=_sc_params,
)
def _segsum(x_hbm, inv_hbm, ab_hbm, osum_hbm,
            dbuf2, nbuf2, ibuf2, abuf, tsum,
            semd, semi):
    cid = lax.axis_index("c")
    sid = lax.axis_index("s")
    wid = sid * 2 + cid

    pltpu.sync_copy(ab_hbm, abuf)
    a_regs, b_regs = _load_coeffs(abuf)

    zrows = nbuf2.at[0]
    _zero_rows(zrows, BLK, TW)
    for j in range(STRIPE // BLK):
        pltpu.sync_copy(zrows,
                        tsum.at[pl.ds(sid * STRIPE + j * BLK, BLK)])
    plsc.subcore_barrier()

    bufs = [(dbuf2.at[p], nbuf2.at[p], ibuf2.at[p], semd.at[p], semi.at[p])
            for p in range(2)]

    for p in range(2):
        b = wid + p * NTILES
        dbuf, nbuf, ibuf, sd, si = bufs[p]

        @pl.when(b < NBLK)
        def _():
            _issue_loads(x_hbm, inv_hbm, b, dbuf, ibuf, sd, si)

    def _phase(k, p):
        b = wid + (k + p) * NTILES
        dbuf, nbuf, ibuf, sd, si = bufs[p]

        @pl.when(b < NBLK)
        def _():
            _wait_loads(x_hbm, inv_hbm, b, dbuf, ibuf, sd, si)
            _normalize_block_cnt(dbuf, nbuf, a_regs, b_regs)
            pltpu.sync_copy(nbuf, tsum.at[ibuf], add=True)
            bn = b + 2 * NTILES

            @pl.when(bn < NBLK)
            def _():
                _issue_loads(x_hbm, inv_hbm, bn, dbuf, ibuf, sd, si)

    @pl.loop(0, 80, step=2)
    def _(k):
        _phase(k, 0)
        _phase(k, 1)

    plsc.subcore_barrier()
    pltpu.sync_copy(tsum.at[pl.ds(sid * STRIPE, STRIPE)],
                    osum_hbm.at[cid, pl.ds(sid * STRIPE, STRIPE)])


MSTR = NSEG_PAD // NTILES


@functools.partial(
    pl.kernel,
    mesh=_sc_mesh,
    out_type=jax.ShapeDtypeStruct((NSEG_PAD, D_OUT), jnp.float32),
    scratch_types=[
        pltpu.VMEM((2, MSTR, TW), jnp.float32),
        pltpu.VMEM((MSTR, D_OUT), jnp.float32),
    ],
    compiler_params=_sc_params,
)
def _means(psum_hbm, o_hbm, sbuf, obuf):
    cid = lax.axis_index("c")
    sid = lax.axis_index("s")
    wid = sid * 2 + cid
    base = wid * MSTR
    for h in range(2):
        pltpu.sync_copy(psum_hbm.at[h, pl.ds(base, MSTR)], sbuf.at[h])

    one = jnp.full((16,), 1.0, jnp.float32)

    @plsc.parallel_loop(0, MSTR, unroll=4)
    def _(r):
        cnt = sbuf[0, r, pl.ds(D_OUT, 16)] + sbuf[1, r, pl.ds(D_OUT, 16)]
        recip = one / jnp.maximum(cnt, one)
        for c in range(4):
            s = (sbuf[0, r, pl.ds(16 * c, 16)]
                 + sbuf[1, r, pl.ds(16 * c, 16)])
            obuf[r, pl.ds(16 * c, 16)] = s * recip

    pltpu.sync_copy(obuf, o_hbm.at[pl.ds(base, MSTR)])


def _out_slices(out_hbm, b, col):
    lo = out_hbm.at[pl.ds(b * BLKH, BLKH), pl.ds(col, D_OUT)]
    hi = out_hbm.at[pl.ds(NH + b * BLKH, BLKH), pl.ds(col, D_OUT)]
    return lo, hi


@functools.partial(
    pl.kernel,
    mesh=_sc_mesh,
    out_type=jax.ShapeDtypeStruct((N, DW), jnp.float32),
    scratch_types=[
        pltpu.VMEM((2, BLKH, DW), jnp.float32),
        pltpu.VMEM((2, BLK, D_OUT), jnp.float32),
        pltpu.VMEM((2, BLK, D_OUT), jnp.float32),
        pltpu.VMEM((2, BLK), jnp.int32),
        pltpu.VMEM((2, DW), jnp.float32),
        pltpu.VMEM_SHARED((NSEG_PAD, D_OUT), jnp.float32),
        pltpu.SemaphoreType.DMA((2,)),
        pltpu.SemaphoreType.DMA((2,)),
        pltpu.SemaphoreType.DMA((2,)),
        pltpu.SemaphoreType.DMA((2,)),
    ],
    compiler_params=_sc_params,
)
def _gather_out(x_hbm, inv_hbm, ab_hbm, means_hbm, out_hbm,
                dbuf2, lbuf2, gbuf2, ibuf2, abuf, smeans,
                semd, semi, semg, semw):
    cid = lax.axis_index("c")
    sid = lax.axis_index("s")
    wid = sid * 2 + cid

    pltpu.sync_copy(means_hbm.at[pl.ds(sid * STRIPE, STRIPE)],
                    smeans.at[pl.ds(sid * STRIPE, STRIPE)])
    pltpu.sync_copy(ab_hbm, abuf)
    a_regs, b_regs = _load_coeffs(abuf)
    plsc.subcore_barrier()

    bufs = [(dbuf2.at[p], lbuf2.at[p], gbuf2.at[p], ibuf2.at[p],
             semd.at[p], semi.at[p], semg.at[p], semw.at[p])
            for p in range(2)]

    for p in range(2):
        b = wid + p * NTILES
        dbuf, lbuf, gbuf, ibuf, sd, si, sg, sw = bufs[p]

        @pl.when(b < NBLK)
        def _():
            _issue_loads(x_hbm, inv_hbm, b, dbuf, ibuf, sd, si)

    def _wait_writes(p, b):
        dbuf, lbuf, gbuf, ibuf, sd, si, sg, sw = bufs[p]
        lo, hi = _out_slices(out_hbm, b, 0)
        glo, ghi = _out_slices(out_hbm, b, D_OUT)
        pltpu.make_async_copy(lbuf.at[pl.ds(0, BLKH)], lo, sw).wait()
        pltpu.make_async_copy(lbuf.at[pl.ds(BLKH, BLKH)], hi, sw).wait()
        pltpu.make_async_copy(gbuf.at[pl.ds(0, BLKH)], glo, sw).wait()
        pltpu.make_async_copy(gbuf.at[pl.ds(BLKH, BLKH)], ghi, sw).wait()

    def _phase(k, p):
        b = wid + (k + p) * NTILES
        dbuf, lbuf, gbuf, ibuf, sd, si, sg, sw = bufs[p]

        @pl.when(b < NBLK)
        def _():
            _wait_loads(x_hbm, inv_hbm, b, dbuf, ibuf, sd, si)

            @pl.when(k + p >= 2)
            def _():
                _wait_writes(p, b)

            pltpu.async_copy(smeans.at[ibuf], gbuf, sg)
            _normalize_block(dbuf, lbuf, a_regs, b_regs)
            lo, hi = _out_slices(out_hbm, b, 0)
            pltpu.async_copy(lbuf.at[pl.ds(0, BLKH)], lo, sw)
            pltpu.async_copy(lbuf.at[pl.ds(BLKH, BLKH)], hi, sw)
            pltpu.make_async_copy(smeans.at[ibuf], gbuf, sg).wait()
            glo, ghi = _out_slices(out_hbm, b, D_OUT)
            pltpu.async_copy(gbuf.at[pl.ds(0, BLKH)], glo, sw)
            pltpu.async_copy(gbuf.at[pl.ds(BLKH, BLKH)], ghi, sw)
            bn = b + 2 * NTILES

            @pl.when(bn < NBLK)
            def _():
                _issue_loads(x_hbm, inv_hbm, bn, dbuf, ibuf, sd, si)

    @pl.loop(0, 80, step=2)
    def _(k):
        _phase(k, 0)
        _phase(k, 1)

    for p in range(2):
        b = wid + p * NTILES

        @pl.when(b < NBLK)
        def _():
            _wait_writes(p, b)


def kernel(inputs, unq_inv, W, gamma, beta):
    wt = W.T
    g2 = gamma.reshape(1, D_OUT)
    b2 = beta.reshape(1, D_OUT)
    x, ab = _matmul_coeffs(inputs, wt, g2, b2)
    psum = _segsum(x, unq_inv, ab)
    means = _means(psum)
    return _gather_out(x, unq_inv, ab, means)
